# baseline (speedup 1.0000x reference)
import functools

import jax
import jax.numpy as jnp
from jax import lax
from jax.experimental import pallas as pl
from jax.experimental.pallas import tpu as pltpu
from jax.experimental.pallas import tpu_sc as plsc

_NC = 2
_NS = 16
_L = 16


def _sc_body(x_hbm, w_hbm, dl_hbm, out_hbm, loss_hbm,
             xb0, xb1, ob0, ob1, wb_v, dl_v, lbf_v,
             isem0, isem1, osem0, osem1, *, B, M, N, F, CB):
    nw = _NC * _NS
    bpw = B // nw
    wid = lax.axis_index("s") * _NC + lax.axis_index("c")
    base = wid * bpw
    nchunks = bpw // CB
    nfc = F // _L
    MN = M * N
    xbufs, obufs = [xb0, xb1], [ob0, ob1]
    isems, osems = [isem0, isem1], [osem0, osem1]

    def in_copies(ci, par):
        return [
            pltpu.make_async_copy(
                x_hbm.at[m, pl.ds(base + ci * CB, CB), :],
                xbufs[par].at[m],
                isems[par],
            )
            for m in range(M)
        ]

    def out_copy(ci, par):
        return pltpu.make_async_copy(
            obufs[par], out_hbm.at[pl.ds(base + ci * CB, CB)], osems[par])

    for c in in_copies(0, 0):
        c.start()
    pltpu.sync_copy(w_hbm.at[pl.ds(base * MN, bpw * MN)], wb_v)
    pltpu.sync_copy(dl_hbm, dl_v)

    def chunk_pair(io, carry):
        for par in range(2):
            ci = io * 2 + par

            @pl.when(ci + 1 < nchunks)
            def _():
                for c in in_copies(ci + 1, 1 - par):
                    c.start()

            for c in in_copies(ci, par):
                c.wait()

            @pl.when(ci >= 2)
            def _():
                out_copy(ci - 2, par).wait()

            xb, ob = xbufs[par], obufs[par]
            for cb in range(CB):
                wbase = (ci * CB + cb) * MN
                wlo = wb_v[pl.ds(wbase, _L)]
                whi = wb_v[pl.ds(wbase + MN - _L, _L)]
                wv = [
                    jnp.full((_L,), wlo[j]) if j < _L
                    else jnp.full((_L,), whi[j - (MN - _L)])
                    for j in range(MN)
                ]

                def f_body(c, carry2):
                    for u in range(2):
                        col = pl.ds((c * 2 + u) * _L, _L)
                        acc = [jnp.zeros((_L,), jnp.float32) for _ in range(N)]
                        for m in range(M):
                            xv = xb[m, cb, col]
                            for n in range(N):
                                acc[n] = acc[n] + wv[m * N + n] * xv
                        for n in range(N):
                            ob[cb, n, col] = acc[n]
                    return carry2

                lax.fori_loop(0, nfc // 2, f_body, 0)
            out_copy(ci, par).start()
        return carry

    lax.fori_loop(0, nchunks // 2, chunk_pair, 0)
    out_copy(nchunks - 2, 0).wait()
    out_copy(nchunks - 1, 1).wait()

    iota = lax.iota(jnp.int32, _L)

    d0 = dl_v[pl.ds(0, _L)]
    d1 = dl_v[pl.ds(_L, _L)]
    d2 = dl_v[pl.ds(2 * _L, _L)]

    def dl_at(j):
        if j < _L:
            return d0[j]
        if j < 2 * _L:
            return d1[j - _L]
        return d2[j - 2 * _L]

    def loss_body(g, carry):
        rows = g * _L + iota
        for n in range(N):
            lacc = jnp.zeros((_L,), jnp.float32)
            for m in range(M):
                j = m * N + n
                dv = jnp.full((_L,), dl_at(j))
                lv = jnp.full((_L,), dl_at(MN + j))
                wvm = plsc.load_gather(wb_v, [rows * MN + j])
                lacc = lacc + wvm * dv + lv
            plsc.store_scatter(lbf_v, [rows * N + n], lacc)
        return carry

    lax.fori_loop(0, bpw // _L, loss_body, 0)
    pltpu.sync_copy(lbf_v, loss_hbm.at[pl.ds(base * N, bpw * N)])


def kernel(x, extra_loss, logits):
    B, M, F = x.shape
    N = logits.shape[1]
    probs = jax.nn.sigmoid(logits)
    wf = jax.random.bernoulli(
        jax.random.key(42), jnp.tile(probs.reshape(-1), B), shape=(B * M * N,)
    ).astype(jnp.float32)
    ls = jax.nn.log_sigmoid(logits)
    lsn = jax.nn.log_sigmoid(-logits)
    dl = jnp.concatenate(
        [jnp.stack([ls - lsn, lsn]).reshape(2 * M * N), jnp.zeros(8, jnp.float32)]
    )

    CB = 4
    mesh = plsc.VectorSubcoreMesh(
        core_axis_name="c", subcore_axis_name="s", num_cores=_NC, num_subcores=_NS
    )
    bpw = B // (_NC * _NS)
    fn = pl.kernel(
        functools.partial(_sc_body, B=B, M=M, N=N, F=F, CB=CB),
        out_type=(
            jax.ShapeDtypeStruct((B, N, F), jnp.float32),
            jax.ShapeDtypeStruct((B * N,), jnp.float32),
        ),
        mesh=mesh,
        compiler_params=pltpu.CompilerParams(
            needs_layout_passes=False, use_tc_tiling_on_sc=True
        ),
        scratch_types=[
            pltpu.VMEM((M, CB, F), jnp.float32),
            pltpu.VMEM((M, CB, F), jnp.float32),
            pltpu.VMEM((CB, N, F), jnp.float32),
            pltpu.VMEM((CB, N, F), jnp.float32),
            pltpu.VMEM((bpw * M * N,), jnp.float32),
            pltpu.VMEM((2 * M * N + 8,), jnp.float32),
            pltpu.VMEM((bpw * N,), jnp.float32),
            pltpu.SemaphoreType.DMA,
            pltpu.SemaphoreType.DMA,
            pltpu.SemaphoreType.DMA,
            pltpu.SemaphoreType.DMA,
        ],
    )
    out, sc_loss = fn(jnp.transpose(x, (1, 0, 2)), wf, dl)
    return (out, extra_loss + sc_loss.reshape(B, N))


# speedup vs baseline: 1.0511x; 1.0511x over previous
import functools

import jax
import jax.numpy as jnp
from jax import lax
from jax.experimental import pallas as pl
from jax.experimental.pallas import tpu as pltpu
from jax.experimental.pallas import tpu_sc as plsc

_NC = 2
_NS = 16
_L = 16


def _sc_body(x_hbm, w_hbm, dl_hbm, out_hbm, loss_hbm,
             xb0, xb1, ob0, ob1, wb_v, dl_v, lbf_v,
             isem0, isem1, osem0, osem1, *, B, M, N, F, CB):
    nw = _NC * _NS
    bpw = B // nw
    wid = lax.axis_index("s") * _NC + lax.axis_index("c")
    base = wid * bpw
    nchunks = bpw // CB
    nfc = F // _L
    MN = M * N
    xbufs, obufs = [xb0, xb1], [ob0, ob1]
    isems, osems = [isem0, isem1], [osem0, osem1]

    def in_copies(ci, par):
        return [
            pltpu.make_async_copy(
                x_hbm.at[:, pl.ds(base + ci * CB, CB), :],
                xbufs[par],
                isems[par],
            )
        ]

    def out_copy(ci, par):
        return pltpu.make_async_copy(
            obufs[par], out_hbm.at[pl.ds(base + ci * CB, CB)], osems[par])

    for c in in_copies(0, 0):
        c.start()
    pltpu.sync_copy(w_hbm.at[pl.ds(base * MN, bpw * MN)], wb_v)
    pltpu.sync_copy(dl_hbm, dl_v)

    def chunk_pair(io, carry):
        for par in range(2):
            ci = io * 2 + par

            @pl.when(ci + 1 < nchunks)
            def _():
                for c in in_copies(ci + 1, 1 - par):
                    c.start()

            for c in in_copies(ci, par):
                c.wait()

            @pl.when(ci >= 2)
            def _():
                out_copy(ci - 2, par).wait()

            xb, ob = xbufs[par], obufs[par]
            for cb in range(CB):
                wbase = (ci * CB + cb) * MN
                wlo = wb_v[pl.ds(wbase, _L)]
                whi = wb_v[pl.ds(wbase + MN - _L, _L)]
                wv = [
                    jnp.full((_L,), wlo[j]) if j < _L
                    else jnp.full((_L,), whi[j - (MN - _L)])
                    for j in range(MN)
                ]

                def f_body(c, carry2):
                    col = pl.ds(c * _L, _L)
                    acc = [jnp.zeros((_L,), jnp.float32) for _ in range(N)]
                    for m in range(M):
                        xv = xb[m, cb, col]
                        for n in range(N):
                            acc[n] = acc[n] + wv[m * N + n] * xv
                    for n in range(N):
                        ob[cb, n, col] = acc[n]
                    return carry2

                lax.fori_loop(0, nfc, f_body, 0)
            out_copy(ci, par).start()
        return carry

    lax.fori_loop(0, nchunks // 2, chunk_pair, 0)
    out_copy(nchunks - 2, 0).wait()
    out_copy(nchunks - 1, 1).wait()

    iota = lax.iota(jnp.int32, _L)

    d0 = dl_v[pl.ds(0, _L)]
    d1 = dl_v[pl.ds(_L, _L)]
    d2 = dl_v[pl.ds(2 * _L, _L)]

    def dl_at(j):
        if j < _L:
            return d0[j]
        if j < 2 * _L:
            return d1[j - _L]
        return d2[j - 2 * _L]

    def loss_body(g, carry):
        rows = g * _L + iota
        for n in range(N):
            lacc = jnp.zeros((_L,), jnp.float32)
            for m in range(M):
                j = m * N + n
                dv = jnp.full((_L,), dl_at(j))
                lv = jnp.full((_L,), dl_at(MN + j))
                wvm = plsc.load_gather(wb_v, [rows * MN + j])
                lacc = lacc + wvm * dv + lv
            plsc.store_scatter(lbf_v, [rows * N + n], lacc)
        return carry

    lax.fori_loop(0, bpw // _L, loss_body, 0)
    pltpu.sync_copy(lbf_v, loss_hbm.at[pl.ds(base * N, bpw * N)])


def kernel(x, extra_loss, logits):
    B, M, F = x.shape
    N = logits.shape[1]
    probs = jax.nn.sigmoid(logits)
    wf = jax.random.bernoulli(
        jax.random.key(42), jnp.tile(probs.reshape(-1), B), shape=(B * M * N,)
    ).astype(jnp.float32)
    ls = jax.nn.log_sigmoid(logits)
    lsn = jax.nn.log_sigmoid(-logits)
    dl = jnp.concatenate(
        [jnp.stack([ls - lsn, lsn]).reshape(2 * M * N), jnp.zeros(8, jnp.float32)]
    )

    CB = 4
    mesh = plsc.VectorSubcoreMesh(
        core_axis_name="c", subcore_axis_name="s", num_cores=_NC, num_subcores=_NS
    )
    bpw = B // (_NC * _NS)
    fn = pl.kernel(
        functools.partial(_sc_body, B=B, M=M, N=N, F=F, CB=CB),
        out_type=(
            jax.ShapeDtypeStruct((B, N, F), jnp.float32),
            jax.ShapeDtypeStruct((B * N,), jnp.float32),
        ),
        mesh=mesh,
        compiler_params=pltpu.CompilerParams(
            needs_layout_passes=False, use_tc_tiling_on_sc=True
        ),
        scratch_types=[
            pltpu.VMEM((M, CB, F), jnp.float32),
            pltpu.VMEM((M, CB, F), jnp.float32),
            pltpu.VMEM((CB, N, F), jnp.float32),
            pltpu.VMEM((CB, N, F), jnp.float32),
            pltpu.VMEM((bpw * M * N,), jnp.float32),
            pltpu.VMEM((2 * M * N + 8,), jnp.float32),
            pltpu.VMEM((bpw * N,), jnp.float32),
            pltpu.SemaphoreType.DMA,
            pltpu.SemaphoreType.DMA,
            pltpu.SemaphoreType.DMA,
            pltpu.SemaphoreType.DMA,
        ],
    )
    out, sc_loss = fn(jnp.transpose(x, (1, 0, 2)), wf, dl)
    return (out, extra_loss + sc_loss.reshape(B, N))
